# Initial kernel scaffold; baseline (speedup 1.0000x reference)
import functools

import jax
import jax.numpy as jnp
from jax import lax
from jax.experimental import pallas as pl
from jax.experimental.pallas import tpu as pltpu
from jax.experimental.pallas import tpu_sc as plsc

N_NODES = 10000
N_EDGES = 320000
FEAT = 16
HID = 128
PACK = 64

NC = 2
NS = 16
NW = NC * NS
EPW = N_EDGES // NW
IROW = 100
RPW = EPW // IROW
CROWS = 10
CH = CROWS * IROW
NCHUNK = RPW // CROWS
TROWS = N_NODES // NS
ZROWS = 125


def _prepass_body(l1, l2, l3, f1, f2, f3, o):
    w1 = jnp.exp(l1[...])
    w2 = jnp.exp(l2[...])
    w3 = jnp.exp(l3[...])
    o[:, 0:16] = w1 * f1[...]
    o[:, 16:32] = w2 * f2[...]
    o[:, 32:48] = w3 * f3[...]
    pad = jnp.zeros((w1.shape[0], 13), jnp.float32)
    o[:, 48:64] = jnp.concatenate([w1, w2, w3, pad], axis=1)


def _prepass(l1, l2, l3, f1, f2, f3):
    be = 6400
    grid = (N_EDGES // be,)
    lspec = pl.BlockSpec((be, 1), lambda j: (j, 0))
    fspec = pl.BlockSpec((be, FEAT), lambda j: (j, 0))
    ospec = pl.BlockSpec((be, PACK), lambda j: (j, 0))
    return pl.pallas_call(
        _prepass_body,
        grid=grid,
        in_specs=[lspec, lspec, lspec, fspec, fspec, fspec],
        out_specs=ospec,
        out_shape=jax.ShapeDtypeStruct((N_EDGES, PACK), jnp.float32),
    )(l1, l2, l3, f1, f2, f3)


def _scatter_body(p_hbm, idx_hbm, out_hbm, acc, idx_v, src_v, stage_v):
    c = lax.axis_index("c")
    s = lax.axis_index("s")
    wid = s * NC + c

    def zrow(i, _):
        z = jnp.zeros((16,), jnp.float32)
        for j in range(PACK // 16):
            stage_v[i, j * 16:(j + 1) * 16] = z
        return 0

    lax.fori_loop(0, ZROWS, zrow, 0)
    for k in range(TROWS // ZROWS):
        pltpu.sync_copy(stage_v, acc.at[pl.ds(s * TROWS + k * ZROWS, ZROWS)])
    plsc.subcore_barrier()

    def chunk(t, _):
        pltpu.sync_copy(idx_hbm.at[wid, pl.ds(t * CROWS, CROWS)], idx_v)
        pltpu.sync_copy(p_hbm.at[pl.ds(wid * EPW + t * CH, CH)], src_v)
        for j in range(CROWS):
            pltpu.sync_copy(src_v.at[pl.ds(j * IROW, IROW)],
                            acc.at[idx_v.at[j]], add=True)
        return 0

    lax.fori_loop(0, NCHUNK, chunk, 0)
    plsc.subcore_barrier()

    for k in range(TROWS // ZROWS):
        rows = pl.ds(s * TROWS + k * ZROWS, ZROWS)
        pltpu.sync_copy(acc.at[rows], stage_v)
        pltpu.sync_copy(stage_v, out_hbm.at[c, rows])


def _scatter(p, idx3):
    mesh = plsc.VectorSubcoreMesh(core_axis_name="c", subcore_axis_name="s",
                                  num_cores=NC, num_subcores=NS)
    fn = pl.kernel(
        _scatter_body,
        out_type=jax.ShapeDtypeStruct((NC, N_NODES, PACK), jnp.float32),
        mesh=mesh,
        scratch_types=[
            pltpu.VMEM_SHARED((N_NODES, PACK), jnp.float32),
            pltpu.VMEM((CROWS, IROW), jnp.int32),
            pltpu.VMEM((CH, PACK), jnp.float32),
            pltpu.VMEM((ZROWS, PACK), jnp.float32),
        ],
    )
    return fn(p, idx3)


def _elu(x):
    return jnp.where(x > 0, x, jnp.expm1(x))


def _finish_body(a0, a1, n1, n2, n3,
                 w1t, b1, w2t, b2, w3t, b3,
                 wat, ba, wnt, bn, wiht, bih, whht, bhh, o):
    acc = a0[...] + a1[...]
    eps = 1e-16

    def ctx(i):
        wt = (w1t, w2t, w3t)[i]
        b = (b1, b2, b3)[i]
        A = acc[:, i * 16:(i + 1) * 16]
        S = acc[:, 48 + i:49 + i]
        c = (jnp.dot(A, wt[...], preferred_element_type=jnp.float32)
             + S * b[...]) / (S + eps)
        return _elu(c)

    context = jnp.concatenate([ctx(0), ctx(1), ctx(2)], axis=1)
    context = jnp.dot(context, wat[...], preferred_element_type=jnp.float32) + ba[...]
    nf = jnp.concatenate([n1[...], n2[...], n3[...]], axis=1)
    nf = jnp.dot(nf, wnt[...], preferred_element_type=jnp.float32) + bn[...]
    gi = jnp.dot(context, wiht[...], preferred_element_type=jnp.float32) + bih[...]
    gh = jnp.dot(nf, whht[...], preferred_element_type=jnp.float32) + bhh[...]
    r = jax.nn.sigmoid(gi[:, 0:HID] + gh[:, 0:HID])
    z = jax.nn.sigmoid(gi[:, HID:2 * HID] + gh[:, HID:2 * HID])
    ng = jnp.tanh(gi[:, 2 * HID:3 * HID] + r * gh[:, 2 * HID:3 * HID])
    h = (1.0 - z) * ng + z * nf
    o[...] = jnp.maximum(h, 0.0)


def _finish(acc, n1, n2, n3, weights):
    bn_ = 2000
    grid = (N_NODES // bn_,)
    aspec = pl.BlockSpec((bn_, PACK), lambda j: (j, 0))
    nspec = pl.BlockSpec((bn_, HID), lambda j: (j, 0))

    def wspec(shape):
        if len(shape) == 1:
            return pl.BlockSpec(shape, lambda j: (0,))
        return pl.BlockSpec(shape, lambda j: (0, 0))

    ws = list(weights)
    in_specs = [aspec, aspec, nspec, nspec, nspec]
    in_specs += [wspec(w.shape) for w in ws]
    return pl.pallas_call(
        _finish_body,
        grid=grid,
        in_specs=in_specs,
        out_specs=pl.BlockSpec((bn_, HID), lambda j: (j, 0)),
        out_shape=jax.ShapeDtypeStruct((N_NODES, HID), jnp.float32),
    )(acc[0], acc[1], n1, n2, n3, *ws)


@jax.jit
def kernel(edge_index, edge_logits1, edge_logits2, edge_logits3,
           edge_feats1, edge_feats2, edge_feats3,
           node_feats1, node_feats2, node_feats3,
           W1, b1, W2, b2, W3, b3, Wa, ba, Wn, bn,
           W_ih, b_ih, W_hh, b_hh):
    p = _prepass(edge_logits1, edge_logits2, edge_logits3,
                 edge_feats1, edge_feats2, edge_feats3)
    idx3 = edge_index[1].reshape(NW, RPW, IROW)
    acc = _scatter(p, idx3)
    weights = (W1.T, b1, W2.T, b2, W3.T, b3, Wa.T, ba, Wn.T, bn,
               W_ih.T, b_ih, W_hh.T, b_hh)
    return _finish(acc, node_feats1, node_feats2, node_feats3, weights)


# speedup vs baseline: 9.3208x; 9.3208x over previous
import functools

import jax
import jax.numpy as jnp
from jax import lax
from jax.experimental import pallas as pl
from jax.experimental.pallas import tpu as pltpu
from jax.experimental.pallas import tpu_sc as plsc

N_NODES = 10000
N_EDGES = 320000
FEAT = 16
HID = 128
PACK = 128

NC = 2
NS = 16
NW = NC * NS
IROW = 128
E_PAD = 327680
EPW = E_PAD // NW
NCHUNK = EPW // IROW
NP = 10240
TROWS = NP // NS
ZROWS = 64


def _prepass_body(l1, l2, l3, f1, f2, f3, o):
    w1 = jnp.exp(l1[...])
    w2 = jnp.exp(l2[...])
    w3 = jnp.exp(l3[...])
    o[:, 0:16] = w1 * f1[...]
    o[:, 16:32] = w2 * f2[...]
    o[:, 32:48] = w3 * f3[...]
    pad = jnp.zeros((w1.shape[0], 13), jnp.float32)
    o[:, 48:64] = jnp.concatenate([w1, w2, w3, pad], axis=1)
    o[:, 64:128] = jnp.zeros((w1.shape[0], 64), jnp.float32)


def _prepass(l1, l2, l3, f1, f2, f3):
    be = 4096
    grid = (E_PAD // be,)
    lspec = pl.BlockSpec((be, 1), lambda j: (j, 0))
    fspec = pl.BlockSpec((be, FEAT), lambda j: (j, 0))
    ospec = pl.BlockSpec((be, PACK), lambda j: (j, 0))
    return pl.pallas_call(
        _prepass_body,
        grid=grid,
        in_specs=[lspec, lspec, lspec, fspec, fspec, fspec],
        out_specs=ospec,
        out_shape=jax.ShapeDtypeStruct((E_PAD, PACK), jnp.float32),
    )(l1, l2, l3, f1, f2, f3)


def _scatter_body(p_hbm, idx_hbm, out_hbm, acc, idx_v, src_v, stage_v):
    c = lax.axis_index("c")
    s = lax.axis_index("s")
    wid = s * NC + c

    def zrow(i, _):
        z = jnp.zeros((16,), jnp.float32)
        for j in range(PACK // 16):
            stage_v[i, j * 16:(j + 1) * 16] = z
        return 0

    lax.fori_loop(0, ZROWS, zrow, 0)
    for k in range(TROWS // ZROWS):
        pltpu.sync_copy(stage_v, acc.at[pl.ds(s * TROWS + k * ZROWS, ZROWS)])
    plsc.subcore_barrier()

    def chunk(t, _):
        pltpu.sync_copy(idx_hbm.at[wid, t], idx_v)
        pltpu.sync_copy(p_hbm.at[pl.ds(wid * EPW + t * IROW, IROW)], src_v)
        pltpu.sync_copy(src_v, acc.at[idx_v.at[0]], add=True)
        return 0

    lax.fori_loop(0, NCHUNK, chunk, 0)
    plsc.subcore_barrier()

    for k in range(TROWS // ZROWS):
        rows = pl.ds(s * TROWS + k * ZROWS, ZROWS)
        pltpu.sync_copy(acc.at[rows], stage_v)
        pltpu.sync_copy(stage_v, out_hbm.at[c, rows])


def _scatter(p, idx3):
    mesh = plsc.VectorSubcoreMesh(core_axis_name="c", subcore_axis_name="s",
                                  num_cores=NC, num_subcores=NS)
    fn = pl.kernel(
        _scatter_body,
        out_type=jax.ShapeDtypeStruct((NC, NP, PACK), jnp.float32),
        mesh=mesh,
        scratch_types=[
            pltpu.VMEM_SHARED((NP, PACK), jnp.float32),
            pltpu.VMEM((1, IROW), jnp.int32),
            pltpu.VMEM((IROW, PACK), jnp.float32),
            pltpu.VMEM((ZROWS, PACK), jnp.float32),
        ],
    )
    return fn(p, idx3)


def _elu(x):
    return jnp.where(x > 0, x, jnp.exp(jnp.minimum(x, 0.0)) - 1.0)


def _finish_body(a0, a1, n1, n2, n3,
                 w1t, b1, w2t, b2, w3t, b3,
                 wat, ba, wnt, bn, wiht, bih, whht, bhh, o):
    acc = a0[...] + a1[...]
    eps = 1e-16

    def ctx(i):
        wt = (w1t, w2t, w3t)[i]
        b = (b1, b2, b3)[i]
        A = acc[:, i * 16:(i + 1) * 16]
        S = acc[:, 48 + i:49 + i]
        c = (jnp.dot(A, wt[...], preferred_element_type=jnp.float32)
             + S * b[...]) / (S + eps)
        return _elu(c)

    context = jnp.concatenate([ctx(0), ctx(1), ctx(2)], axis=1)
    context = jnp.dot(context, wat[...], preferred_element_type=jnp.float32) + ba[...]
    nf = jnp.concatenate([n1[...], n2[...], n3[...]], axis=1)
    nf = jnp.dot(nf, wnt[...], preferred_element_type=jnp.float32) + bn[...]
    gi = jnp.dot(context, wiht[...], preferred_element_type=jnp.float32) + bih[...]
    gh = jnp.dot(nf, whht[...], preferred_element_type=jnp.float32) + bhh[...]
    r = jax.nn.sigmoid(gi[:, 0:HID] + gh[:, 0:HID])
    z = jax.nn.sigmoid(gi[:, HID:2 * HID] + gh[:, HID:2 * HID])
    ng = jnp.tanh(gi[:, 2 * HID:3 * HID] + r * gh[:, 2 * HID:3 * HID])
    h = (1.0 - z) * ng + z * nf
    o[...] = jnp.maximum(h, 0.0)


def _finish(acc, n1, n2, n3, weights):
    bn_ = 2000
    grid = (N_NODES // bn_,)
    aspec = pl.BlockSpec((bn_, PACK), lambda j: (j, 0))
    nspec = pl.BlockSpec((bn_, HID), lambda j: (j, 0))

    def wspec(shape):
        if len(shape) == 1:
            return pl.BlockSpec(shape, lambda j: (0,))
        return pl.BlockSpec(shape, lambda j: (0, 0))

    ws = list(weights)
    in_specs = [aspec, aspec, nspec, nspec, nspec]
    in_specs += [wspec(w.shape) for w in ws]
    return pl.pallas_call(
        _finish_body,
        grid=grid,
        in_specs=in_specs,
        out_specs=pl.BlockSpec((bn_, HID), lambda j: (j, 0)),
        out_shape=jax.ShapeDtypeStruct((N_NODES, HID), jnp.float32),
    )(acc[0], acc[1], n1, n2, n3, *ws)


@jax.jit
def kernel(edge_index, edge_logits1, edge_logits2, edge_logits3,
           edge_feats1, edge_feats2, edge_feats3,
           node_feats1, node_feats2, node_feats3,
           W1, b1, W2, b2, W3, b3, Wa, ba, Wn, bn,
           W_ih, b_ih, W_hh, b_hh):
    npad = E_PAD - N_EDGES
    pl_ = lambda x: jnp.pad(x, ((0, npad), (0, 0)), constant_values=-1e30)
    pf_ = lambda x: jnp.pad(x, ((0, npad), (0, 0)))
    p = _prepass(pl_(edge_logits1), pl_(edge_logits2), pl_(edge_logits3),
                 pf_(edge_feats1), pf_(edge_feats2), pf_(edge_feats3))
    dstp = jnp.pad(edge_index[1], (0, npad))
    idx4 = dstp.reshape(NW, NCHUNK, 1, IROW)
    acc = _scatter(p, idx4)
    weights = (W1.T, b1, W2.T, b2, W3.T, b3, Wa.T, ba, Wn.T, bn,
               W_ih.T, b_ih, W_hh.T, b_hh)
    return _finish(acc, node_feats1, node_feats2, node_feats3, weights)


# speedup vs baseline: 12.7496x; 1.3679x over previous
import functools

import jax
import jax.numpy as jnp
from jax import lax
from jax.experimental import pallas as pl
from jax.experimental.pallas import tpu as pltpu
from jax.experimental.pallas import tpu_sc as plsc

N_NODES = 10000
N_EDGES = 320000
FEAT = 16
HID = 128
PACK = 128

NC = 2
NS = 16
NW = NC * NS
IROW = 128
EPW = N_EDGES // NW
NFULL = EPW // IROW
TAIL_OFF = EPW - IROW
NCHUNK = NFULL + 1
DUMMY = 10239
NP = 10240
TROWS = NP // NS
ZROWS = 64


def _prepass_body(l1, l2, l3, f1, f2, f3, o):
    w1 = jnp.exp(l1[...])
    w2 = jnp.exp(l2[...])
    w3 = jnp.exp(l3[...])
    o[:, 0:16] = w1 * f1[...]
    o[:, 16:32] = w2 * f2[...]
    o[:, 32:48] = w3 * f3[...]
    pad = jnp.zeros((w1.shape[0], 13), jnp.float32)
    o[:, 48:64] = jnp.concatenate([w1, w2, w3, pad], axis=1)
    o[:, 64:128] = jnp.zeros((w1.shape[0], 64), jnp.float32)


def _prepass(l1, l2, l3, f1, f2, f3):
    be = 4000
    grid = (N_EDGES // be,)
    lspec = pl.BlockSpec((be, 1), lambda j: (j, 0))
    fspec = pl.BlockSpec((be, FEAT), lambda j: (j, 0))
    ospec = pl.BlockSpec((be, PACK), lambda j: (j, 0))
    return pl.pallas_call(
        _prepass_body,
        grid=grid,
        in_specs=[lspec, lspec, lspec, fspec, fspec, fspec],
        out_specs=ospec,
        out_shape=jax.ShapeDtypeStruct((N_EDGES, PACK), jnp.float32),
    )(l1, l2, l3, f1, f2, f3)


def _scatter_body(p_hbm, idx_hbm, out_hbm, acc, idx_v, src_v, stage_v):
    c = lax.axis_index("c")
    s = lax.axis_index("s")
    wid = s * NC + c

    def zrow(i, _):
        z = jnp.zeros((16,), jnp.float32)
        for j in range(PACK // 16):
            stage_v[i, j * 16:(j + 1) * 16] = z
        return 0

    lax.fori_loop(0, ZROWS, zrow, 0)
    for k in range(TROWS // ZROWS):
        pltpu.sync_copy(stage_v, acc.at[pl.ds(s * TROWS + k * ZROWS, ZROWS)])
    plsc.subcore_barrier()

    def chunk(t, _):
        off = pl.multiple_of(wid * EPW + jnp.minimum(t * IROW, TAIL_OFF), 8)
        pltpu.sync_copy(idx_hbm.at[wid, t], idx_v)
        pltpu.sync_copy(p_hbm.at[pl.ds(off, IROW)], src_v)
        pltpu.sync_copy(src_v, acc.at[idx_v.at[0]], add=True)
        return 0

    lax.fori_loop(0, NCHUNK, chunk, 0)
    plsc.subcore_barrier()

    for k in range(TROWS // ZROWS):
        rows = pl.ds(s * TROWS + k * ZROWS, ZROWS)
        pltpu.sync_copy(acc.at[rows], stage_v)
        pltpu.sync_copy(stage_v, out_hbm.at[c, rows])


def _scatter(p, idx3):
    mesh = plsc.VectorSubcoreMesh(core_axis_name="c", subcore_axis_name="s",
                                  num_cores=NC, num_subcores=NS)
    fn = pl.kernel(
        _scatter_body,
        out_type=jax.ShapeDtypeStruct((NC, NP, PACK), jnp.float32),
        mesh=mesh,
        scratch_types=[
            pltpu.VMEM_SHARED((NP, PACK), jnp.float32),
            pltpu.VMEM((1, IROW), jnp.int32),
            pltpu.VMEM((IROW, PACK), jnp.float32),
            pltpu.VMEM((ZROWS, PACK), jnp.float32),
        ],
    )
    return fn(p, idx3)


def _elu(x):
    return jnp.where(x > 0, x, jnp.exp(jnp.minimum(x, 0.0)) - 1.0)


def _finish_body(a0, a1, n1, n2, n3,
                 w1t, b1, w2t, b2, w3t, b3,
                 wat, ba, wnt, bn, wiht, bih, whht, bhh, o):
    acc = a0[...] + a1[...]
    eps = 1e-16

    def ctx(i):
        wt = (w1t, w2t, w3t)[i]
        b = (b1, b2, b3)[i]
        A = acc[:, i * 16:(i + 1) * 16]
        S = acc[:, 48 + i:49 + i]
        c = (jnp.dot(A, wt[...], preferred_element_type=jnp.float32)
             + S * b[...]) / (S + eps)
        return _elu(c)

    context = jnp.concatenate([ctx(0), ctx(1), ctx(2)], axis=1)
    context = jnp.dot(context, wat[...], preferred_element_type=jnp.float32) + ba[...]
    nf = jnp.concatenate([n1[...], n2[...], n3[...]], axis=1)
    nf = jnp.dot(nf, wnt[...], preferred_element_type=jnp.float32) + bn[...]
    gi = jnp.dot(context, wiht[...], preferred_element_type=jnp.float32) + bih[...]
    gh = jnp.dot(nf, whht[...], preferred_element_type=jnp.float32) + bhh[...]
    r = jax.nn.sigmoid(gi[:, 0:HID] + gh[:, 0:HID])
    z = jax.nn.sigmoid(gi[:, HID:2 * HID] + gh[:, HID:2 * HID])
    ng = jnp.tanh(gi[:, 2 * HID:3 * HID] + r * gh[:, 2 * HID:3 * HID])
    h = (1.0 - z) * ng + z * nf
    o[...] = jnp.maximum(h, 0.0)


def _finish(acc, n1, n2, n3, weights):
    bn_ = 2000
    grid = (N_NODES // bn_,)
    aspec = pl.BlockSpec((bn_, PACK), lambda j: (j, 0))
    nspec = pl.BlockSpec((bn_, HID), lambda j: (j, 0))

    def wspec(shape):
        if len(shape) == 1:
            return pl.BlockSpec(shape, lambda j: (0,))
        return pl.BlockSpec(shape, lambda j: (0, 0))

    ws = list(weights)
    in_specs = [aspec, aspec, nspec, nspec, nspec]
    in_specs += [wspec(w.shape) for w in ws]
    return pl.pallas_call(
        _finish_body,
        grid=grid,
        in_specs=in_specs,
        out_specs=pl.BlockSpec((bn_, HID), lambda j: (j, 0)),
        out_shape=jax.ShapeDtypeStruct((N_NODES, HID), jnp.float32),
    )(acc[0], acc[1], n1, n2, n3, *ws)


@jax.jit
def kernel(edge_index, edge_logits1, edge_logits2, edge_logits3,
           edge_feats1, edge_feats2, edge_feats3,
           node_feats1, node_feats2, node_feats3,
           W1, b1, W2, b2, W3, b3, Wa, ba, Wn, bn,
           W_ih, b_ih, W_hh, b_hh):
    p = _prepass(edge_logits1, edge_logits2, edge_logits3,
                 edge_feats1, edge_feats2, edge_feats3)
    base = edge_index[1].reshape(NW, EPW)
    head = base[:, :NFULL * IROW].reshape(NW, NFULL, IROW)
    ndup = IROW - (EPW - NFULL * IROW)
    tail = jnp.concatenate(
        [jnp.full((NW, ndup), DUMMY, jnp.int32),
         base[:, NFULL * IROW:]], axis=1).reshape(NW, 1, IROW)
    idx4 = jnp.concatenate([head, tail], axis=1).reshape(NW, NCHUNK, 1, IROW)
    acc = _scatter(p, idx4)
    weights = (W1.T, b1, W2.T, b2, W3.T, b3, Wa.T, ba, Wn.T, bn,
               W_ih.T, b_ih, W_hh.T, b_hh)
    return _finish(acc, node_feats1, node_feats2, node_feats3, weights)


# speedup vs baseline: 14.8247x; 1.1628x over previous
import functools

import jax
import jax.numpy as jnp
from jax import lax
from jax.experimental import pallas as pl
from jax.experimental.pallas import tpu as pltpu
from jax.experimental.pallas import tpu_sc as plsc

N_NODES = 10000
N_EDGES = 320000
FEAT = 16
HID = 128
PACK = 128

NC = 2
NS = 16
NW = NC * NS
IROW = 128
EPW = N_EDGES // NW
NFULL = EPW // IROW
TAIL_OFF = EPW - IROW
NCHUNK = NFULL + 1
DUMMY = 10239
NP = 10240
TROWS = NP // NS
ZROWS = 64


NB = 10
EB = N_EDGES // 8 // NB
LB = EB // 16


def _prepass_body(l1, l2, l3, f1, f2, f3, g, h, rmat, mmat, o):
    gm = g[0]
    hm = h[0]
    rm = rmat[...]
    mm = mmat[...]
    b = pl.program_id(1)

    def wcol(lref):
        lb = lref[pl.ds(b * LB, 256), :]
        t = jnp.dot(jnp.exp(lb), gm, preferred_element_type=jnp.float32)
        u = jnp.dot(rm, t, preferred_element_type=jnp.float32)
        return jnp.sum(u * mm, axis=1, keepdims=True)

    def fsel(fref):
        return jnp.dot(fref[...], hm, preferred_element_type=jnp.float32)

    w1, w2, w3 = wcol(l1), wcol(l2), wcol(l3)
    o[...] = jnp.concatenate(
        [fsel(f1) * w1, fsel(f2) * w2, fsel(f3) * w3,
         w1, w2, w3, jnp.zeros((EB, PACK - 51), jnp.float32)], axis=1)


def _prepass(l1, l2, l3, f1, f2, f3, g, h, rmat, mmat):
    grid = (8, NB)
    lspec = pl.BlockSpec((N_EDGES // 128 + 60, 128), lambda j, b: (0, 0))
    fspec = pl.BlockSpec((EB, 128), lambda j, b: (b, 0))
    sspec = pl.BlockSpec((1, 128, 16), lambda j, b: (j, 0, 0))
    rspec = pl.BlockSpec((EB, 256), lambda j, b: (0, 0))
    mspec = pl.BlockSpec((EB, 16), lambda j, b: (0, 0))
    ospec = pl.BlockSpec((EB, PACK), lambda j, b: (j * NB + b, 0))
    return pl.pallas_call(
        _prepass_body,
        grid=grid,
        in_specs=[lspec, lspec, lspec, fspec, fspec, fspec, sspec, sspec,
                  rspec, mspec],
        out_specs=ospec,
        out_shape=jax.ShapeDtypeStruct((N_EDGES, PACK), jnp.float32),
    )(l1, l2, l3, f1, f2, f3, g, h, rmat, mmat)


def _scatter_body(p_hbm, idx_hbm, out_hbm, acc, idx_v, src_v, stage_v):
    c = lax.axis_index("c")
    s = lax.axis_index("s")
    wid = s * NC + c

    def zrow(i, _):
        z = jnp.zeros((16,), jnp.float32)
        for j in range(PACK // 16):
            stage_v[i, j * 16:(j + 1) * 16] = z
        return 0

    lax.fori_loop(0, ZROWS, zrow, 0)
    for k in range(TROWS // ZROWS):
        pltpu.sync_copy(stage_v, acc.at[pl.ds(s * TROWS + k * ZROWS, ZROWS)])
    plsc.subcore_barrier()

    def chunk(t, _):
        off = pl.multiple_of(wid * EPW + jnp.minimum(t * IROW, TAIL_OFF), 8)
        pltpu.sync_copy(idx_hbm.at[wid, t], idx_v)
        pltpu.sync_copy(p_hbm.at[pl.ds(off, IROW)], src_v)
        pltpu.sync_copy(src_v, acc.at[idx_v.at[0]], add=True)
        return 0

    lax.fori_loop(0, NCHUNK, chunk, 0)
    plsc.subcore_barrier()

    for k in range(TROWS // ZROWS):
        rows = pl.ds(s * TROWS + k * ZROWS, ZROWS)
        pltpu.sync_copy(acc.at[rows], stage_v)
        pltpu.sync_copy(stage_v, out_hbm.at[c, rows])


def _scatter(p, idx3):
    mesh = plsc.VectorSubcoreMesh(core_axis_name="c", subcore_axis_name="s",
                                  num_cores=NC, num_subcores=NS)
    fn = pl.kernel(
        _scatter_body,
        out_type=jax.ShapeDtypeStruct((NC, NP, PACK), jnp.float32),
        mesh=mesh,
        scratch_types=[
            pltpu.VMEM_SHARED((NP, PACK), jnp.float32),
            pltpu.VMEM((1, IROW), jnp.int32),
            pltpu.VMEM((IROW, PACK), jnp.float32),
            pltpu.VMEM((ZROWS, PACK), jnp.float32),
        ],
    )
    return fn(p, idx3)


def _elu(x):
    return jnp.where(x > 0, x, jnp.exp(jnp.minimum(x, 0.0)) - 1.0)


def _finish_body(a0, a1, n1, n2, n3,
                 w1t, b1, w2t, b2, w3t, b3,
                 wat, ba, wnt, bn, wiht, bih, whht, bhh, o):
    acc = a0[...] + a1[...]
    eps = 1e-16

    def ctx(i):
        wt = (w1t, w2t, w3t)[i]
        b = (b1, b2, b3)[i]
        A = acc[:, i * 16:(i + 1) * 16]
        S = acc[:, 48 + i:49 + i]
        c = (jnp.dot(A, wt[...], preferred_element_type=jnp.float32)
             + S * b[...]) / (S + eps)
        return _elu(c)

    context = jnp.concatenate([ctx(0), ctx(1), ctx(2)], axis=1)
    context = jnp.dot(context, wat[...], preferred_element_type=jnp.float32) + ba[...]
    nf = jnp.concatenate([n1[...], n2[...], n3[...]], axis=1)
    nf = jnp.dot(nf, wnt[...], preferred_element_type=jnp.float32) + bn[...]
    gi = jnp.dot(context, wiht[...], preferred_element_type=jnp.float32) + bih[...]
    gh = jnp.dot(nf, whht[...], preferred_element_type=jnp.float32) + bhh[...]
    r = jax.nn.sigmoid(gi[:, 0:HID] + gh[:, 0:HID])
    z = jax.nn.sigmoid(gi[:, HID:2 * HID] + gh[:, HID:2 * HID])
    ng = jnp.tanh(gi[:, 2 * HID:3 * HID] + r * gh[:, 2 * HID:3 * HID])
    h = (1.0 - z) * ng + z * nf
    o[...] = jnp.maximum(h, 0.0)


def _finish(acc, n1, n2, n3, weights):
    bn_ = 2000
    grid = (N_NODES // bn_,)
    aspec = pl.BlockSpec((bn_, PACK), lambda j: (j, 0))
    nspec = pl.BlockSpec((bn_, HID), lambda j: (j, 0))

    def wspec(shape):
        if len(shape) == 1:
            return pl.BlockSpec(shape, lambda j: (0,))
        return pl.BlockSpec(shape, lambda j: (0, 0))

    ws = list(weights)
    in_specs = [aspec, aspec, nspec, nspec, nspec]
    in_specs += [wspec(w.shape) for w in ws]
    return pl.pallas_call(
        _finish_body,
        grid=grid,
        in_specs=in_specs,
        out_specs=pl.BlockSpec((bn_, HID), lambda j: (j, 0)),
        out_shape=jax.ShapeDtypeStruct((N_NODES, HID), jnp.float32),
    )(acc[0], acc[1], n1, n2, n3, *ws)


@jax.jit
def kernel(edge_index, edge_logits1, edge_logits2, edge_logits3,
           edge_feats1, edge_feats2, edge_feats3,
           node_feats1, node_feats2, node_feats3,
           W1, b1, W2, b2, W3, b3, Wa, ba, Wn, bn,
           W_ih, b_ih, W_hh, b_hh):
    lanes = jnp.arange(128)[None, :, None]
    cols = jnp.arange(16)[None, None, :]
    res = jnp.arange(8)[:, None, None]
    g = (lanes == 8 * cols + res).astype(jnp.float32)
    h = (lanes == 16 * res + cols).astype(jnp.float32)
    rmat = (jnp.arange(EB)[:, None] // 16 ==
            jnp.arange(256)[None, :]).astype(jnp.float32)
    mmat = (jnp.arange(EB)[:, None] % 16 ==
            jnp.arange(16)[None, :]).astype(jnp.float32)
    ld = lambda x: jnp.pad(x.reshape(N_EDGES // 128, 128), ((0, 60), (0, 0)))
    p = _prepass(ld(edge_logits1), ld(edge_logits2), ld(edge_logits3),
                 edge_feats1.reshape(N_EDGES // 8, 128),
                 edge_feats2.reshape(N_EDGES // 8, 128),
                 edge_feats3.reshape(N_EDGES // 8, 128),
                 g, h, rmat, mmat)
    dst_sigma = edge_index[1].reshape(N_EDGES // 8, 8).T.reshape(-1)
    base = dst_sigma.reshape(NW, EPW)
    head = base[:, :NFULL * IROW].reshape(NW, NFULL, IROW)
    ndup = IROW - (EPW - NFULL * IROW)
    tail = jnp.concatenate(
        [jnp.full((NW, ndup), DUMMY, jnp.int32),
         base[:, NFULL * IROW:]], axis=1).reshape(NW, 1, IROW)
    idx4 = jnp.concatenate([head, tail], axis=1).reshape(NW, NCHUNK, 1, IROW)
    acc = _scatter(p, idx4)
    weights = (W1.T, b1, W2.T, b2, W3.T, b3, Wa.T, ba, Wn.T, bn,
               W_ih.T, b_ih, W_hh.T, b_hh)
    return _finish(acc, node_feats1, node_feats2, node_feats3, weights)


# speedup vs baseline: 16.5990x; 1.1197x over previous
import functools

import jax
import jax.numpy as jnp
from jax import lax
from jax.experimental import pallas as pl
from jax.experimental.pallas import tpu as pltpu
from jax.experimental.pallas import tpu_sc as plsc

N_NODES = 10000
N_EDGES = 320000
FEAT = 16
HID = 128
PACK = 128

NC = 2
NS = 16
NW = NC * NS
IROW = 128
EPW = N_EDGES // NW
NFULL = EPW // IROW
TAIL_OFF = EPW - IROW
NCHUNK = NFULL + 1
DUMMY = 10239
NP = 10240
TROWS = NP // NS
ZROWS = 64


NB = 10
EB = N_EDGES // 8 // NB
LB = EB // 16


def _prepass_body(l1, l2, l3, f1, f2, f3, g, h, rmat, mmat, o):
    gm = g[0]
    hm = h[0]
    rm = rmat[...]
    mm = mmat[...]
    b = pl.program_id(1)

    def wcol(lref):
        lb = lref[pl.ds(b * LB, 256), :]
        t = jnp.dot(jnp.exp(lb), gm, preferred_element_type=jnp.float32)
        u = jnp.dot(rm, t, preferred_element_type=jnp.float32)
        return jnp.sum(u * mm, axis=1, keepdims=True)

    def fsel(fref):
        return jnp.dot(fref[...], hm, preferred_element_type=jnp.float32)

    w1, w2, w3 = wcol(l1), wcol(l2), wcol(l3)
    o[...] = jnp.concatenate(
        [fsel(f1) * w1, fsel(f2) * w2, fsel(f3) * w3,
         w1, w2, w3, jnp.zeros((EB, PACK - 51), jnp.float32)], axis=1)


def _prepass(l1, l2, l3, f1, f2, f3, g, h, rmat, mmat):
    grid = (8, NB)
    lspec = pl.BlockSpec((N_EDGES // 128 + 60, 128), lambda j, b: (0, 0))
    fspec = pl.BlockSpec((EB, 128), lambda j, b: (b, 0))
    sspec = pl.BlockSpec((1, 128, 16), lambda j, b: (j, 0, 0))
    rspec = pl.BlockSpec((EB, 256), lambda j, b: (0, 0))
    mspec = pl.BlockSpec((EB, 16), lambda j, b: (0, 0))
    ospec = pl.BlockSpec((EB, PACK), lambda j, b: (j * NB + b, 0))
    return pl.pallas_call(
        _prepass_body,
        grid=grid,
        in_specs=[lspec, lspec, lspec, fspec, fspec, fspec, sspec, sspec,
                  rspec, mspec],
        out_specs=ospec,
        out_shape=jax.ShapeDtypeStruct((N_EDGES, PACK), jnp.float32),
    )(l1, l2, l3, f1, f2, f3, g, h, rmat, mmat)


def _scatter_body(p_hbm, idx_hbm, out_hbm, acc,
                  idx_a, src_a, idx_b, src_b, stage_v, sem_a, sem_b):
    c = lax.axis_index("c")
    s = lax.axis_index("s")
    wid = s * NC + c

    def zrow(i, _):
        z = jnp.zeros((16,), jnp.float32)
        for j in range(PACK // 16):
            stage_v[i, j * 16:(j + 1) * 16] = z
        return 0

    lax.fori_loop(0, ZROWS, zrow, 0)
    for k in range(TROWS // ZROWS):
        pltpu.sync_copy(stage_v, acc.at[pl.ds(s * TROWS + k * ZROWS, ZROWS)])
    plsc.subcore_barrier()

    def off(t):
        return pl.multiple_of(wid * EPW + jnp.minimum(t * IROW, TAIL_OFF), 8)

    def start(t, ib, sb, sem):
        pltpu.async_copy(idx_hbm.at[wid, t], ib, sem)
        pltpu.async_copy(p_hbm.at[pl.ds(off(t), IROW)], sb, sem)

    def drain_scat(t0, ib, sb, sem):
        pltpu.make_async_copy(idx_hbm.at[wid, t0], ib, sem).wait()
        pltpu.make_async_copy(p_hbm.at[pl.ds(off(t0), IROW)], sb, sem).wait()
        pltpu.sync_copy(sb, acc.at[ib.at[0]], add=True)

    start(0, idx_a, src_a, sem_a)

    def pair(k, _):
        t = 2 * k
        start(t + 1, idx_b, src_b, sem_b)
        drain_scat(t, idx_a, src_a, sem_a)
        start(t + 2, idx_a, src_a, sem_a)
        drain_scat(t + 1, idx_b, src_b, sem_b)
        return 0

    lax.fori_loop(0, (NCHUNK - 1) // 2, pair, 0)
    drain_scat(NCHUNK - 1, idx_a, src_a, sem_a)
    plsc.subcore_barrier()

    for k in range(TROWS // ZROWS):
        rows = pl.ds(s * TROWS + k * ZROWS, ZROWS)
        pltpu.sync_copy(acc.at[rows], stage_v)
        pltpu.sync_copy(stage_v, out_hbm.at[c, rows])


def _scatter(p, idx3):
    mesh = plsc.VectorSubcoreMesh(core_axis_name="c", subcore_axis_name="s",
                                  num_cores=NC, num_subcores=NS)
    fn = pl.kernel(
        _scatter_body,
        out_type=jax.ShapeDtypeStruct((NC, NP, PACK), jnp.float32),
        mesh=mesh,
        scratch_types=[
            pltpu.VMEM_SHARED((NP, PACK), jnp.float32),
            pltpu.VMEM((1, IROW), jnp.int32),
            pltpu.VMEM((IROW, PACK), jnp.float32),
            pltpu.VMEM((1, IROW), jnp.int32),
            pltpu.VMEM((IROW, PACK), jnp.float32),
            pltpu.VMEM((ZROWS, PACK), jnp.float32),
            pltpu.SemaphoreType.DMA,
            pltpu.SemaphoreType.DMA,
        ],
    )
    return fn(p, idx3)


def _elu(x):
    return jnp.where(x > 0, x, jnp.exp(jnp.minimum(x, 0.0)) - 1.0)


def _finish_body(a0, a1, n1, n2, n3,
                 w1t, b1, w2t, b2, w3t, b3,
                 wat, ba, wnt, bn, wiht, bih, whht, bhh, o):
    acc = a0[...] + a1[...]
    eps = 1e-16

    def ctx(i):
        wt = (w1t, w2t, w3t)[i]
        b = (b1, b2, b3)[i]
        A = acc[:, i * 16:(i + 1) * 16]
        S = acc[:, 48 + i:49 + i]
        c = (jnp.dot(A, wt[...], preferred_element_type=jnp.float32)
             + S * b[...]) / (S + eps)
        return _elu(c)

    context = jnp.concatenate([ctx(0), ctx(1), ctx(2)], axis=1)
    context = jnp.dot(context, wat[...], preferred_element_type=jnp.float32) + ba[...]
    nf = jnp.concatenate([n1[...], n2[...], n3[...]], axis=1)
    nf = jnp.dot(nf, wnt[...], preferred_element_type=jnp.float32) + bn[...]
    gi = jnp.dot(context, wiht[...], preferred_element_type=jnp.float32) + bih[...]
    gh = jnp.dot(nf, whht[...], preferred_element_type=jnp.float32) + bhh[...]
    r = jax.nn.sigmoid(gi[:, 0:HID] + gh[:, 0:HID])
    z = jax.nn.sigmoid(gi[:, HID:2 * HID] + gh[:, HID:2 * HID])
    ng = jnp.tanh(gi[:, 2 * HID:3 * HID] + r * gh[:, 2 * HID:3 * HID])
    h = (1.0 - z) * ng + z * nf
    o[...] = jnp.maximum(h, 0.0)


def _finish(acc, n1, n2, n3, weights):
    bn_ = 2000
    grid = (N_NODES // bn_,)
    aspec = pl.BlockSpec((bn_, PACK), lambda j: (j, 0))
    nspec = pl.BlockSpec((bn_, HID), lambda j: (j, 0))

    def wspec(shape):
        if len(shape) == 1:
            return pl.BlockSpec(shape, lambda j: (0,))
        return pl.BlockSpec(shape, lambda j: (0, 0))

    ws = list(weights)
    in_specs = [aspec, aspec, nspec, nspec, nspec]
    in_specs += [wspec(w.shape) for w in ws]
    return pl.pallas_call(
        _finish_body,
        grid=grid,
        in_specs=in_specs,
        out_specs=pl.BlockSpec((bn_, HID), lambda j: (j, 0)),
        out_shape=jax.ShapeDtypeStruct((N_NODES, HID), jnp.float32),
    )(acc[0], acc[1], n1, n2, n3, *ws)


@jax.jit
def kernel(edge_index, edge_logits1, edge_logits2, edge_logits3,
           edge_feats1, edge_feats2, edge_feats3,
           node_feats1, node_feats2, node_feats3,
           W1, b1, W2, b2, W3, b3, Wa, ba, Wn, bn,
           W_ih, b_ih, W_hh, b_hh):
    lanes = jnp.arange(128)[None, :, None]
    cols = jnp.arange(16)[None, None, :]
    res = jnp.arange(8)[:, None, None]
    g = (lanes == 8 * cols + res).astype(jnp.float32)
    h = (lanes == 16 * res + cols).astype(jnp.float32)
    rmat = (jnp.arange(EB)[:, None] // 16 ==
            jnp.arange(256)[None, :]).astype(jnp.float32)
    mmat = (jnp.arange(EB)[:, None] % 16 ==
            jnp.arange(16)[None, :]).astype(jnp.float32)
    ld = lambda x: jnp.pad(x.reshape(N_EDGES // 128, 128), ((0, 60), (0, 0)))
    p = _prepass(ld(edge_logits1), ld(edge_logits2), ld(edge_logits3),
                 edge_feats1.reshape(N_EDGES // 8, 128),
                 edge_feats2.reshape(N_EDGES // 8, 128),
                 edge_feats3.reshape(N_EDGES // 8, 128),
                 g, h, rmat, mmat)
    dst_sigma = edge_index[1].reshape(N_EDGES // 8, 8).T.reshape(-1)
    base = dst_sigma.reshape(NW, EPW)
    head = base[:, :NFULL * IROW].reshape(NW, NFULL, IROW)
    ndup = IROW - (EPW - NFULL * IROW)
    tail = jnp.concatenate(
        [jnp.full((NW, ndup), DUMMY, jnp.int32),
         base[:, NFULL * IROW:]], axis=1).reshape(NW, 1, IROW)
    idx4 = jnp.concatenate([head, tail], axis=1).reshape(NW, NCHUNK, 1, IROW)
    acc = _scatter(p, idx4)
    weights = (W1.T, b1, W2.T, b2, W3.T, b3, Wa.T, ba, Wn.T, bn,
               W_ih.T, b_ih, W_hh.T, b_hh)
    return _finish(acc, node_feats1, node_feats2, node_feats3, weights)


# speedup vs baseline: 16.7838x; 1.0111x over previous
import functools

import jax
import jax.numpy as jnp
import numpy as np
from jax import lax
from jax.experimental import pallas as pl
from jax.experimental.pallas import tpu as pltpu
from jax.experimental.pallas import tpu_sc as plsc

N_NODES = 10000
N_EDGES = 320000
FEAT = 16
HID = 128
PACK = 128

NC = 2
NS = 16
NW = NC * NS
IROW = 128
EPW = N_EDGES // NW
NFULL = EPW // IROW
TAIL_OFF = EPW - IROW
NCHUNK = NFULL + 1
DUMMY = 10239
NP = 10240
TROWS = NP // NS
ZROWS = 64


NB = 10
EB = N_EDGES // 8 // NB
LB = EB // 16

_lanes = np.arange(128)[None, :, None]
_cols = np.arange(16)[None, None, :]
_res = np.arange(8)[:, None, None]
_G_CONST = (_lanes == 8 * _cols + _res).astype(np.float32)
_H_CONST = (_lanes == 16 * _res + _cols).astype(np.float32)
_R_CONST = (np.arange(EB)[:, None] // 16 ==
            np.arange(256)[None, :]).astype(np.float32)
_M_CONST = (np.arange(EB)[:, None] % 16 ==
            np.arange(16)[None, :]).astype(np.float32)


def _prepass_body(l1, l2, l3, f1, f2, f3, g, h, rmat, mmat, o):
    gm = g[0]
    hm = h[0]
    rm = rmat[...]
    mm = mmat[...]
    b = pl.program_id(1)

    def wcol(lref):
        lb = lref[pl.ds(b * LB, 256), :]
        t = jnp.dot(jnp.exp(lb), gm, preferred_element_type=jnp.float32)
        u = jnp.dot(rm, t, preferred_element_type=jnp.float32)
        return jnp.sum(u * mm, axis=1, keepdims=True)

    def fsel(fref):
        return jnp.dot(fref[...], hm, preferred_element_type=jnp.float32)

    w1, w2, w3 = wcol(l1), wcol(l2), wcol(l3)
    o[...] = jnp.concatenate(
        [fsel(f1) * w1, fsel(f2) * w2, fsel(f3) * w3,
         w1, w2, w3, jnp.zeros((EB, PACK - 51), jnp.float32)], axis=1)


def _prepass(l1, l2, l3, f1, f2, f3, g, h, rmat, mmat):
    grid = (8, NB)
    lspec = pl.BlockSpec((N_EDGES // 128 + 60, 128), lambda j, b: (0, 0))
    fspec = pl.BlockSpec((EB, 128), lambda j, b: (b, 0))
    sspec = pl.BlockSpec((1, 128, 16), lambda j, b: (j, 0, 0))
    rspec = pl.BlockSpec((EB, 256), lambda j, b: (0, 0))
    mspec = pl.BlockSpec((EB, 16), lambda j, b: (0, 0))
    ospec = pl.BlockSpec((EB, PACK), lambda j, b: (j * NB + b, 0))
    return pl.pallas_call(
        _prepass_body,
        grid=grid,
        in_specs=[lspec, lspec, lspec, fspec, fspec, fspec, sspec, sspec,
                  rspec, mspec],
        out_specs=ospec,
        out_shape=jax.ShapeDtypeStruct((N_EDGES, PACK), jnp.float32),
    )(l1, l2, l3, f1, f2, f3, g, h, rmat, mmat)


def _scatter_body(p_hbm, idx_hbm, out_hbm, acc,
                  idx_a, src_a, idx_b, src_b, stage_v, sem_a, sem_b):
    c = lax.axis_index("c")
    s = lax.axis_index("s")
    wid = s * NC + c

    def zrow(i, _):
        z = jnp.zeros((16,), jnp.float32)
        for j in range(PACK // 16):
            stage_v[i, j * 16:(j + 1) * 16] = z
        return 0

    lax.fori_loop(0, ZROWS, zrow, 0)
    for k in range(TROWS // ZROWS):
        pltpu.sync_copy(stage_v, acc.at[pl.ds(s * TROWS + k * ZROWS, ZROWS)])
    plsc.subcore_barrier()

    def off(t):
        return pl.multiple_of(wid * EPW + jnp.minimum(t * IROW, TAIL_OFF), 8)

    def start(t, ib, sb, sem):
        pltpu.async_copy(idx_hbm.at[wid, t], ib, sem)
        pltpu.async_copy(p_hbm.at[pl.ds(off(t), IROW)], sb, sem)

    def drain_scat(t0, ib, sb, sem):
        pltpu.make_async_copy(idx_hbm.at[wid, t0], ib, sem).wait()
        pltpu.make_async_copy(p_hbm.at[pl.ds(off(t0), IROW)], sb, sem).wait()
        pltpu.sync_copy(sb, acc.at[ib.at[0]], add=True)

    start(0, idx_a, src_a, sem_a)

    def pair(k, _):
        t = 2 * k
        start(t + 1, idx_b, src_b, sem_b)
        drain_scat(t, idx_a, src_a, sem_a)
        start(t + 2, idx_a, src_a, sem_a)
        drain_scat(t + 1, idx_b, src_b, sem_b)
        return 0

    lax.fori_loop(0, (NCHUNK - 1) // 2, pair, 0)
    drain_scat(NCHUNK - 1, idx_a, src_a, sem_a)
    plsc.subcore_barrier()

    for k in range(TROWS // ZROWS):
        rows = pl.ds(s * TROWS + k * ZROWS, ZROWS)
        pltpu.sync_copy(acc.at[rows], stage_v)
        pltpu.sync_copy(stage_v, out_hbm.at[c, rows])


def _scatter(p, idx3):
    mesh = plsc.VectorSubcoreMesh(core_axis_name="c", subcore_axis_name="s",
                                  num_cores=NC, num_subcores=NS)
    fn = pl.kernel(
        _scatter_body,
        out_type=jax.ShapeDtypeStruct((NC, NP, PACK), jnp.float32),
        mesh=mesh,
        scratch_types=[
            pltpu.VMEM_SHARED((NP, PACK), jnp.float32),
            pltpu.VMEM((1, IROW), jnp.int32),
            pltpu.VMEM((IROW, PACK), jnp.float32),
            pltpu.VMEM((1, IROW), jnp.int32),
            pltpu.VMEM((IROW, PACK), jnp.float32),
            pltpu.VMEM((ZROWS, PACK), jnp.float32),
            pltpu.SemaphoreType.DMA,
            pltpu.SemaphoreType.DMA,
        ],
    )
    return fn(p, idx3)


def _elu(x):
    return jnp.where(x > 0, x, jnp.exp(jnp.minimum(x, 0.0)) - 1.0)


def _finish_body(a0, a1, n1, n2, n3,
                 w1t, b1, w2t, b2, w3t, b3,
                 wat, ba, wnt, bn, wiht, bih, whht, bhh, o):
    acc = a0[...] + a1[...]
    eps = 1e-16

    def ctx(i):
        wt = (w1t, w2t, w3t)[i]
        b = (b1, b2, b3)[i]
        A = acc[:, i * 16:(i + 1) * 16]
        S = acc[:, 48 + i:49 + i]
        c = (jnp.dot(A, wt[...], preferred_element_type=jnp.float32)
             + S * b[...]) / (S + eps)
        return _elu(c)

    context = jnp.concatenate([ctx(0), ctx(1), ctx(2)], axis=1)
    context = jnp.dot(context, wat[...], preferred_element_type=jnp.float32) + ba[...]
    nf = jnp.concatenate([n1[...], n2[...], n3[...]], axis=1)
    nf = jnp.dot(nf, wnt[...], preferred_element_type=jnp.float32) + bn[...]
    gi = jnp.dot(context, wiht[...], preferred_element_type=jnp.float32) + bih[...]
    gh = jnp.dot(nf, whht[...], preferred_element_type=jnp.float32) + bhh[...]
    r = jax.nn.sigmoid(gi[:, 0:HID] + gh[:, 0:HID])
    z = jax.nn.sigmoid(gi[:, HID:2 * HID] + gh[:, HID:2 * HID])
    ng = jnp.tanh(gi[:, 2 * HID:3 * HID] + r * gh[:, 2 * HID:3 * HID])
    h = (1.0 - z) * ng + z * nf
    o[...] = jnp.maximum(h, 0.0)


def _finish(acc, n1, n2, n3, weights):
    bn_ = 2000
    grid = (N_NODES // bn_,)
    aspec = pl.BlockSpec((bn_, PACK), lambda j: (j, 0))
    nspec = pl.BlockSpec((bn_, HID), lambda j: (j, 0))

    def wspec(shape):
        if len(shape) == 1:
            return pl.BlockSpec(shape, lambda j: (0,))
        return pl.BlockSpec(shape, lambda j: (0, 0))

    ws = list(weights)
    in_specs = [aspec, aspec, nspec, nspec, nspec]
    in_specs += [wspec(w.shape) for w in ws]
    return pl.pallas_call(
        _finish_body,
        grid=grid,
        in_specs=in_specs,
        out_specs=pl.BlockSpec((bn_, HID), lambda j: (j, 0)),
        out_shape=jax.ShapeDtypeStruct((N_NODES, HID), jnp.float32),
    )(acc[0], acc[1], n1, n2, n3, *ws)


@jax.jit
def kernel(edge_index, edge_logits1, edge_logits2, edge_logits3,
           edge_feats1, edge_feats2, edge_feats3,
           node_feats1, node_feats2, node_feats3,
           W1, b1, W2, b2, W3, b3, Wa, ba, Wn, bn,
           W_ih, b_ih, W_hh, b_hh):
    g = jnp.asarray(_G_CONST)
    h = jnp.asarray(_H_CONST)
    rmat = jnp.asarray(_R_CONST)
    mmat = jnp.asarray(_M_CONST)
    ld = lambda x: jnp.pad(x.reshape(N_EDGES // 128, 128), ((0, 60), (0, 0)))
    p = _prepass(ld(edge_logits1), ld(edge_logits2), ld(edge_logits3),
                 edge_feats1.reshape(N_EDGES // 8, 128),
                 edge_feats2.reshape(N_EDGES // 8, 128),
                 edge_feats3.reshape(N_EDGES // 8, 128),
                 g, h, rmat, mmat)
    dst_sigma = edge_index[1].reshape(N_EDGES // 8, 8).T.reshape(-1)
    base = dst_sigma.reshape(NW, EPW)
    head = base[:, :NFULL * IROW].reshape(NW, NFULL, IROW)
    ndup = IROW - (EPW - NFULL * IROW)
    tail = jnp.concatenate(
        [jnp.full((NW, ndup), DUMMY, jnp.int32),
         base[:, NFULL * IROW:]], axis=1).reshape(NW, 1, IROW)
    idx4 = jnp.concatenate([head, tail], axis=1).reshape(NW, NCHUNK, 1, IROW)
    acc = _scatter(p, idx4)
    weights = (W1.T, b1, W2.T, b2, W3.T, b3, Wa.T, ba, Wn.T, bn,
               W_ih.T, b_ih, W_hh.T, b_hh)
    return _finish(acc, node_feats1, node_feats2, node_feats3, weights)
